# baseline (speedup 1.0000x reference)
import jax
import jax.numpy as jnp
from jax.experimental import pallas as pl


def _mm_body(x_ref, w_ref, o_ref):
    o_ref[...] = jnp.dot(x_ref[...], w_ref[...], preferred_element_type=jnp.float32)


def _matmul(x, w):
    n, k = x.shape
    f = w.shape[1]
    bn = 1000
    return pl.pallas_call(
        _mm_body,
        grid=(n // bn,),
        in_specs=[
            pl.BlockSpec((bn, k), lambda i: (i, 0)),
            pl.BlockSpec((k, f), lambda i: (0, 0)),
        ],
        out_specs=pl.BlockSpec((bn, f), lambda i: (i, 0)),
        out_shape=jax.ShapeDtypeStruct((n, f), jnp.float32),
    )(x, w)


def _gat(x, src, dst, W, a_src, a_dst, b):
    N = x.shape[0]
    h = _matmul(x, W)
    alpha_s = jnp.sum(h * a_src, axis=-1)
    alpha_d = jnp.sum(h * a_dst, axis=-1)
    e = jax.nn.leaky_relu(alpha_s[src] + alpha_d[dst], 0.2)
    e_max = jax.ops.segment_max(e, dst, num_segments=N)
    e_max = jnp.where(jnp.isfinite(e_max), e_max, 0.0)
    ex = jnp.exp(e - e_max[dst])
    denom = jax.ops.segment_sum(ex, dst, num_segments=N)
    alpha = ex / (denom[dst] + 1e-16)
    out = jax.ops.segment_sum(h[src] * alpha[:, None], dst, num_segments=N)
    return out + b


def kernel(x, edge_index, W1, a_src1, a_dst1, b1, W2, a_src2, a_dst2, b2,
           W3, a_src3, a_dst3, b3, W4, a_src4, a_dst4, b4):
    N = x.shape[0]
    loop = jnp.arange(N, dtype=edge_index.dtype)
    src = jnp.concatenate([edge_index[0], loop])
    dst = jnp.concatenate([edge_index[1], loop])
    h = jax.nn.relu(_gat(x, src, dst, W1, a_src1, a_dst1, b1))
    h = jax.nn.relu(_gat(h, src, dst, W2, a_src2, a_dst2, b2))
    h = jax.nn.relu(_gat(h, src, dst, W3, a_src3, a_dst3, b3))
    h4 = _matmul(h, jnp.pad(W4, ((0, 0), (0, 127))))[:, :1]
    alpha_s = jnp.sum(h4 * a_src4, axis=-1)
    alpha_d = jnp.sum(h4 * a_dst4, axis=-1)
    e = jax.nn.leaky_relu(alpha_s[src] + alpha_d[dst], 0.2)
    e_max = jax.ops.segment_max(e, dst, num_segments=N)
    e_max = jnp.where(jnp.isfinite(e_max), e_max, 0.0)
    ex = jnp.exp(e - e_max[dst])
    denom = jax.ops.segment_sum(ex, dst, num_segments=N)
    alpha = ex / (denom[dst] + 1e-16)
    out = jax.ops.segment_sum(h4[src, 0] * alpha, dst, num_segments=N)
    return out + b4[0]


# speedup vs baseline: 3.3350x; 3.3350x over previous
import functools

import jax
import jax.numpy as jnp
from jax import lax
from jax.experimental import pallas as pl
from jax.experimental.pallas import tpu as pltpu
from jax.experimental.pallas import tpu_sc as plsc

_NC = 2
_NS = 16
_NW = _NC * _NS
_L = 16


def _proj_body(y_ref, w_ref, a2_ref, b_ref, h_ref, as_ref, *, relu):
    y = y_ref[...]
    if relu:
        y = jnp.maximum(y + b_ref[...], 0.0)
    h = jnp.dot(y, w_ref[...], preferred_element_type=jnp.float32)
    h_ref[...] = h
    as_ref[...] = jnp.dot(h, a2_ref[...], preferred_element_type=jnp.float32)


def _project(y, w, a2, b, relu):
    n, k = y.shape
    f = w.shape[1]
    bn = 1000
    return pl.pallas_call(
        functools.partial(_proj_body, relu=relu),
        grid=(n // bn,),
        in_specs=[
            pl.BlockSpec((bn, k), lambda i: (i, 0)),
            pl.BlockSpec((k, f), lambda i: (0, 0)),
            pl.BlockSpec((f, 128), lambda i: (0, 0)),
            pl.BlockSpec((1, k), lambda i: (0, 0)),
        ],
        out_specs=[
            pl.BlockSpec((bn, f), lambda i: (i, 0)),
            pl.BlockSpec((bn, 128), lambda i: (i, 0)),
        ],
        out_shape=[
            jax.ShapeDtypeStruct((n, f), jnp.float32),
            jax.ShapeDtypeStruct((n, 128), jnp.float32),
        ],
    )(y, w, a2, b)


def _edge_phase(d, n, n_pad, e_pad, cpw):
    dpw = n_pad // _NW
    ch = dpw // cpw
    nbnd = cpw * _NW + 1
    nbnd_pad = -(-(nbnd + _L) // 8) * 8

    def body(srcs, dsts, bounds, h, als, ald, out,
             als_v, ald_v, m_v, s_v, bnd_v, sb, db, gix, dlv, etmp, atmp,
             rows, obuf, sem):
        wid = lax.axis_index("s") * _NC + lax.axis_index("c")
        dlo = wid * dpw
        iota = lax.iota(jnp.int32, _L)

        pltpu.sync_copy(als, als_v)
        pltpu.sync_copy(ald, ald_v)
        pltpu.sync_copy(bounds, bnd_v)

        for i in range(dpw // _L):
            m_v[pl.ds(_L * i, _L)] = jnp.full((_L,), -1e30, jnp.float32)
            s_v[pl.ds(_L * i, _L)] = jnp.zeros((_L,), jnp.float32)

        def bscal(idx):
            return bnd_v[pl.ds(idx, _L)][0]

        e_lo = bscal(cpw * wid)
        e_hi = bscal(cpw * wid + cpw)
        g0 = (e_lo // _L) * _L
        n_g = (e_hi - g0 + (_L - 1)) // _L

        def load_group(base):
            pltpu.sync_copy(srcs.at[pl.ds(base, _L)], sb)
            pltpu.sync_copy(dsts.at[pl.ds(base, _L)], db)
            srcv = sb[...]
            seg = db[...]
            srcc = jnp.clip(srcv, 0, n - 1)
            segc = jnp.clip(seg, 0, n_pad - 1)
            a_s = plsc.load_gather(als_v, [srcc])
            a_d = plsc.load_gather(ald_v, [segc])
            z = a_s + a_d
            e = jnp.where(z >= 0, z, jnp.float32(0.2) * z)
            return srcc, seg, e

        def seg_scan(seg, val, is_max):
            for k in (1, 2, 4, 8):
                idxk = jnp.maximum(iota - k, 0)
                dprev = plsc.load_gather(db, [idxk])
                etmp[...] = val
                vprev = plsc.load_gather(etmp, [idxk])
                mk = (iota >= k) & (dprev == seg)
                if is_max:
                    val = jnp.where(mk, jnp.maximum(val, vprev), val)
                else:
                    val = val + jnp.where(mk, vprev, jnp.float32(0.0))
            idxn = jnp.minimum(iota + 1, _L - 1)
            dnext = plsc.load_gather(db, [idxn])
            last = (iota == _L - 1) | (dnext != seg)
            return val, last

        def pass_a(gi, carry):
            base = g0 + gi * _L
            _, seg, e = load_group(base)
            valid = (seg >= dlo) & (seg < dlo + dpw)
            runmax, last = seg_scan(seg, e, True)
            lastv = last & valid
            dloc = jnp.clip(seg - dlo, 0, dpw - 1)
            cur = plsc.load_gather(m_v, [dloc], mask=lastv)
            plsc.store_scatter(m_v, [dloc], jnp.maximum(cur, runmax),
                               mask=lastv)
            return carry

        lax.fori_loop(0, n_g, pass_a, 0)

        def pass_b(gi, carry):
            base = g0 + gi * _L
            _, seg, e = load_group(base)
            valid = (seg >= dlo) & (seg < dlo + dpw)
            dloc = jnp.clip(seg - dlo, 0, dpw - 1)
            mval = plsc.load_gather(m_v, [dloc])
            ex = jnp.exp(e - mval)
            ex = jnp.where(valid, ex, jnp.float32(0.0))
            runsum, last = seg_scan(seg, ex, False)
            lastv = last & valid
            plsc.addupdate_scatter(s_v, [dloc], runsum, mask=lastv)
            return carry

        lax.fori_loop(0, n_g, pass_b, 0)

        def chunk(c, ccarry):
            clo = dlo + c * ch
            b_lo = bscal(cpw * wid + c)
            b_hi = bscal(cpw * wid + c + 1)

            def zero_row(r, carry):
                for j in range(d // _L):
                    obuf[r, pl.ds(_L * j, _L)] = jnp.zeros((_L,), jnp.float32)
                return carry

            lax.fori_loop(0, ch, zero_row, 0)

            g0c = (b_lo // _L) * _L
            n_gc = (b_hi - g0c + (_L - 1)) // _L

            def group(gi, carry):
                base = g0c + gi * _L
                srcc, seg, e = load_group(base)
                valid = (seg >= clo) & (seg < clo + ch)
                dloc = jnp.clip(seg - dlo, 0, dpw - 1)
                mval = plsc.load_gather(m_v, [dloc])
                sval = plsc.load_gather(s_v, [dloc])
                alpha = jnp.exp(e - mval) / (sval + jnp.float32(1e-16))
                alpha = jnp.where(valid, alpha, jnp.float32(0.0))
                gix[...] = srcc
                cp = pltpu.async_copy(h.at[gix], rows, sem)
                dl = jnp.clip(seg - clo, 0, ch - 1)
                cp.wait()

                for i in range(_L):
                    a_i = alpha[i]
                    d_i = dl[i]
                    av = lax.broadcast(a_i, (_L,))
                    for j in range(d // _L):
                        plsc.addupdate(obuf.at[d_i, pl.ds(_L * j, _L)],
                                       av * rows[i, pl.ds(_L * j, _L)])
                return carry

            lax.fori_loop(0, n_gc, group, 0)

            @pl.when(clo + ch <= n)
            def _():
                pltpu.sync_copy(obuf, out.at[pl.ds(clo, ch)])

            return ccarry

        lax.fori_loop(0, cpw, chunk, 0)

    mesh = plsc.VectorSubcoreMesh(core_axis_name="c", subcore_axis_name="s")
    return pl.kernel(
        body,
        out_type=jax.ShapeDtypeStruct((n, d), jnp.float32),
        mesh=mesh,
        compiler_params=pltpu.CompilerParams(needs_layout_passes=False),
        scratch_types=[
            pltpu.VMEM((n_pad,), jnp.float32),
            pltpu.VMEM((n_pad,), jnp.float32),
            pltpu.VMEM((dpw,), jnp.float32),
            pltpu.VMEM((dpw,), jnp.float32),
            pltpu.VMEM((nbnd_pad,), jnp.int32),
            pltpu.VMEM((_L,), jnp.int32),
            pltpu.VMEM((_L,), jnp.int32),
            pltpu.VMEM((_L,), jnp.int32),
            pltpu.VMEM((_L,), jnp.int32),
            pltpu.VMEM((_L,), jnp.float32),
            pltpu.VMEM((_L,), jnp.float32),
            pltpu.VMEM((_L, d), jnp.float32),
            pltpu.VMEM((ch, d), jnp.float32),
            pltpu.SemaphoreType.DMA,
        ],
    )


def kernel(x, edge_index, W1, a_src1, a_dst1, b1, W2, a_src2, a_dst2, b2,
           W3, a_src3, a_dst3, b3, W4, a_src4, a_dst4, b4):
    n = x.shape[0]
    e = edge_index.shape[1]
    n_pad = -(-n // (_NW * _L)) * (_NW * _L)
    cpw = 4
    ch = n_pad // (_NW * cpw)
    assert n % ch == 0

    loop = jnp.arange(n, dtype=edge_index.dtype)
    src_full = jnp.concatenate([edge_index[0], loop]).astype(jnp.int32)
    dst_full = jnp.concatenate([edge_index[1], loop]).astype(jnp.int32)
    order = jnp.argsort(dst_full)
    srcs = src_full[order]
    dsts = dst_full[order]
    ne = e + n
    e_pad = -(-ne // _L) * _L
    srcs = jnp.pad(srcs, (0, e_pad - ne))
    dsts = jnp.pad(dsts, (0, e_pad - ne), constant_values=jnp.int32(2 ** 30))
    grid_vals = jnp.arange(0, n_pad + 1, ch, dtype=jnp.int32)
    bounds = jnp.searchsorted(dsts, grid_vals).astype(jnp.int32)
    nbnd_pad = -(-(bounds.shape[0] + _L) // 8) * 8
    bounds = jnp.pad(bounds, (0, nbnd_pad - bounds.shape[0]))

    def pack_a2(f, a_src, a_dst):
        a2 = jnp.zeros((f, 128), jnp.float32)
        a2 = a2.at[: a_src.shape[0], 0].set(a_src)
        a2 = a2.at[: a_dst.shape[0], 1].set(a_dst)
        return a2

    def pad_n(v):
        return jnp.pad(v, (0, n_pad - n))

    edge512 = _edge_phase(512, n, n_pad, e_pad, cpw)
    edge128 = _edge_phase(128, n, n_pad, e_pad, cpw)

    def gat_layer(y, w, a2, b_prev, relu, edge_fn, d):
        h, asd = _project(y, w, a2, b_prev, relu)
        if d != h.shape[1]:
            h = h[:, :d]
        als = pad_n(asd[:, 0])
        ald = pad_n(asd[:, 1])
        return edge_fn(srcs, dsts, bounds, h, als, ald)

    zb = jnp.zeros((1, x.shape[1]), jnp.float32)
    o1 = gat_layer(x, W1, pack_a2(512, a_src1, a_dst1), zb, False, edge512, 512)
    o2 = gat_layer(o1, W2, pack_a2(512, a_src2, a_dst2), b1[None, :], True,
                   edge512, 512)
    o3 = gat_layer(o2, W3, pack_a2(512, a_src3, a_dst3), b2[None, :], True,
                   edge512, 512)
    w4p = jnp.pad(W4, ((0, 0), (0, 127)))
    o4 = gat_layer(o3, w4p, pack_a2(128, a_src4, a_dst4), b3[None, :], True,
                   edge128, 128)
    return o4[:, 0] + b4[0]
